# baseline (speedup 1.0000x reference)
import functools

import jax
import jax.numpy as jnp
from jax import lax
from jax.experimental import pallas as pl
from jax.experimental.pallas import tpu as pltpu
from jax.experimental.pallas import tpu_sc as plsc

N = 8192
KNN = 16
CIN = 256
MID = 256
COUT = 256
SHARE = 8
WDIM = MID // SHARE
EPS = 1e-5

_QKV_BLK = 512
_TOPK_BLK = 256
_FUSE_BLK = 256
_GCHUNK = 128


def _bdot(a, b):
    return jnp.dot(a.astype(jnp.bfloat16), b.astype(jnp.bfloat16),
                   preferred_element_type=jnp.float32)


def _qkv_body(x_ref, wq_ref, bq_ref, wk_ref, bk_ref, wv_ref, bv_ref,
              q_ref, k_ref, v_ref):
    x = x_ref[:].astype(jnp.bfloat16)
    q_ref[:] = jnp.dot(x, wq_ref[:].astype(jnp.bfloat16), preferred_element_type=jnp.float32) + bq_ref[:]
    k_ref[:] = jnp.dot(x, wk_ref[:].astype(jnp.bfloat16), preferred_element_type=jnp.float32) + bk_ref[:]
    v_ref[:] = jnp.dot(x, wv_ref[:].astype(jnp.bfloat16), preferred_element_type=jnp.float32) + bv_ref[:]


def _qkv(x, wqT, bq, wkT, bk, wvT, bv, interpret=False):
    nblk = N // _QKV_BLK
    full = lambda i: (0, 0)
    row = lambda i: (i, 0)
    return pl.pallas_call(
        _qkv_body,
        grid=(nblk,),
        in_specs=[
            pl.BlockSpec((_QKV_BLK, CIN), row),
            pl.BlockSpec((CIN, MID), full),
            pl.BlockSpec((1, MID), full),
            pl.BlockSpec((CIN, MID), full),
            pl.BlockSpec((1, MID), full),
            pl.BlockSpec((CIN, COUT), full),
            pl.BlockSpec((1, COUT), full),
        ],
        out_specs=[
            pl.BlockSpec((_QKV_BLK, MID), row),
            pl.BlockSpec((_QKV_BLK, MID), row),
            pl.BlockSpec((_QKV_BLK, COUT), row),
        ],
        out_shape=[
            jax.ShapeDtypeStruct((N, MID), jnp.float32),
            jax.ShapeDtypeStruct((N, MID), jnp.float32),
            jax.ShapeDtypeStruct((N, COUT), jnp.float32),
        ],
        interpret=interpret,
    )(x, wqT, bq, wkT, bk, wvT, bv)


def _topk_body(p_ref, pT_ref, idx_ref):
    p_blk = p_ref[:]
    px = p_blk[:, 0:1]
    py = p_blk[:, 1:2]
    pz = p_blk[:, 2:3]
    tx = pT_ref[0:1, :]
    ty = pT_ref[1:2, :]
    tz = pT_ref[2:3, :]
    dot = jnp.dot(p_blk.astype(jnp.bfloat16), pT_ref[:].astype(jnp.bfloat16),
                  preferred_element_type=jnp.float32)
    sqb = (px * px + py * py) + pz * pz
    sqa = (tx * tx + ty * ty) + tz * tz
    d2 = (sqb + sqa) - 2.0 * dot

    HW = N // 2
    va = d2[:, :HW]
    vb = d2[:, HW:]
    swap = vb < va
    v0 = jnp.where(swap, vb, va)
    v1 = jnp.where(swap, va, vb)
    s0 = jnp.where(swap, jnp.int32(1), jnp.int32(0))
    cols = lax.broadcasted_iota(jnp.int32, v0.shape, 1)
    inf = jnp.float32(jnp.inf)
    picks = []
    for n in range(KNN):
        j = jnp.argmin(v0, axis=1).reshape(-1, 1).astype(jnp.int32)
        mask = cols == j
        sbit = jnp.max(jnp.where(mask, s0, jnp.int32(0)), axis=1, keepdims=True)
        picks.append(j + HW * sbit)
        if n < KNN - 1:
            v0 = jnp.where(mask, v1, v0)
            s0 = jnp.where(mask, jnp.int32(1) - s0, s0)
            v1 = jnp.where(mask, inf, v1)
    idx_ref[:] = jnp.concatenate(picks, axis=1)


def _topk_qkv_body(p_ref, pT_ref, x_ref, wq_ref, bq_ref, wk_ref, bk_ref,
                   wv_ref, bv_ref, idx_ref, q_ref, k_ref, v_ref):
    x = x_ref[:].astype(jnp.bfloat16)
    q_ref[:] = jnp.dot(x, wq_ref[:].astype(jnp.bfloat16), preferred_element_type=jnp.float32) + bq_ref[:]
    k_ref[:] = jnp.dot(x, wk_ref[:].astype(jnp.bfloat16), preferred_element_type=jnp.float32) + bk_ref[:]
    v_ref[:] = jnp.dot(x, wv_ref[:].astype(jnp.bfloat16), preferred_element_type=jnp.float32) + bv_ref[:]
    _topk_body(p_ref, pT_ref, idx_ref)


def _topk_qkv(p8_rows, pT8, x, wqT, bq, wkT, bk, wvT, bv, interpret=False):
    nrows = p8_rows.shape[0]
    nblk = nrows // _TOPK_BLK
    xblk = N // nblk
    full = lambda i: (0, 0)
    row = lambda i: (i, 0)
    return pl.pallas_call(
        _topk_qkv_body,
        grid=(nblk,),
        in_specs=[
            pl.BlockSpec((_TOPK_BLK, 8), row),
            pl.BlockSpec((8, N), full),
            pl.BlockSpec((xblk, CIN), row),
            pl.BlockSpec((CIN, MID), full),
            pl.BlockSpec((1, MID), full),
            pl.BlockSpec((CIN, MID), full),
            pl.BlockSpec((1, MID), full),
            pl.BlockSpec((CIN, COUT), full),
            pl.BlockSpec((1, COUT), full),
        ],
        out_specs=[
            pl.BlockSpec((_TOPK_BLK, KNN), row),
            pl.BlockSpec((xblk, MID), row),
            pl.BlockSpec((xblk, MID), row),
            pl.BlockSpec((xblk, COUT), row),
        ],
        out_shape=[
            jax.ShapeDtypeStruct((nrows, KNN), jnp.int32),
            jax.ShapeDtypeStruct((N, MID), jnp.float32),
            jax.ShapeDtypeStruct((N, MID), jnp.float32),
            jax.ShapeDtypeStruct((N, COUT), jnp.float32),
        ],
        interpret=interpret,
    )(p8_rows, pT8, x, wqT, bq, wkT, bk, wvT, bv)


def _topk(p_rows, pT8, interpret=False):
    nrows = p_rows.shape[0]
    nblk = nrows // _TOPK_BLK
    return pl.pallas_call(
        _topk_body,
        grid=(nblk,),
        in_specs=[
            pl.BlockSpec((_TOPK_BLK, 8), lambda i: (i, 0)),
            pl.BlockSpec((8, N), lambda i: (0, 0)),
        ],
        out_specs=pl.BlockSpec((_TOPK_BLK, KNN), lambda i: (i, 0)),
        out_shape=jax.ShapeDtypeStruct((nrows, KNN), jnp.int32),
        interpret=interpret,
    )(p_rows, pT8)


def _sc_gather(idx_flat, xk, xv, pfat):
    info = plsc.get_sparse_core_info()
    nw = info.num_cores * info.num_subcores
    total = idx_flat.shape[0]
    per_w = total // nw
    nchunk = per_w // _GCHUNK
    mesh = plsc.VectorSubcoreMesh(core_axis_name="c", subcore_axis_name="s")

    @functools.partial(
        pl.kernel,
        mesh=mesh,
        compiler_params=pltpu.CompilerParams(use_tc_tiling_on_sc=False),
        out_type=[
            jax.ShapeDtypeStruct((total, MID), jnp.float32),
            jax.ShapeDtypeStruct((total, COUT), jnp.float32),
            jax.ShapeDtypeStruct((total, 16), jnp.float32),
        ],
        scratch_types=[
            pltpu.VMEM((_GCHUNK,), jnp.int32),
            pltpu.VMEM((_GCHUNK, MID), jnp.float32),
            pltpu.VMEM((_GCHUNK, COUT), jnp.float32),
            pltpu.VMEM((_GCHUNK, 16), jnp.float32),
            pltpu.SemaphoreType.DMA,
            pltpu.SemaphoreType.DMA,
            pltpu.SemaphoreType.DMA,
        ],
    )
    def gather_kernel(idx_hbm, xk_hbm, xv_hbm, pf_hbm,
                      kg_hbm, vg_hbm, pg_hbm,
                      idx_v, kbuf, vbuf, pbuf, sem_k, sem_v, sem_p):
        wid = lax.axis_index("s") * info.num_cores + lax.axis_index("c")
        base = wid * per_w

        def body(c, carry):
            off = base + c * _GCHUNK
            pltpu.sync_copy(idx_hbm.at[pl.ds(off, _GCHUNK)], idx_v)
            ck = pltpu.async_copy(xk_hbm.at[idx_v], kbuf, sem_k)
            cv = pltpu.async_copy(xv_hbm.at[idx_v], vbuf, sem_v)
            cp = pltpu.async_copy(pf_hbm.at[idx_v], pbuf, sem_p)
            ck.wait()
            cv.wait()
            cp.wait()
            pltpu.sync_copy(kbuf, kg_hbm.at[pl.ds(off, _GCHUNK)])
            pltpu.sync_copy(vbuf, vg_hbm.at[pl.ds(off, _GCHUNK)])
            pltpu.sync_copy(pbuf, pg_hbm.at[pl.ds(off, _GCHUNK)])
            return carry

        lax.fori_loop(0, nchunk, body, 0)

    return gather_kernel(idx_flat, xk, xv, pfat)


def _fuse_body(kg_ref, vg_ref, pg_ref, q_ref, pf_ref,
               a1_ref, c1_ref, a2_ref, c2_ref,
               s1_ref, h1_ref, b1_ref, d1_ref, b2_ref, d2_ref,
               out_ref):
    q = q_ref[:]
    pf = pf_ref[:]
    a1 = a1_ref[:]
    c1 = c1_ref[:]
    a2 = a2_ref[:]
    c2 = c2_ref[:]
    s1 = s1_ref[:]
    h1 = h1_ref[:]
    b1 = b1_ref[:]
    d1 = d1_ref[:]
    b2 = b2_ref[:]
    d2 = d2_ref[:]

    prs = []
    ws = []
    for n in range(KNN):
        pr_in = pg_ref[n] - pf
        t = jnp.dot(pr_in, a1, preferred_element_type=jnp.float32) + c1
        t = jnp.maximum(t, 0.0)
        pr_n = jnp.dot(t, a2, preferred_element_type=jnp.float32) + c2
        prs.append(pr_n)
        wn = (kg_ref[n] - q + pr_n) * s1 + h1
        wn = jnp.maximum(wn, 0.0)
        wn = _bdot(wn, b1) + d1
        wn = jnp.maximum(wn, 0.0)
        wn = _bdot(wn, b2) + d2
        ws.append(wn)

    m = ws[0]
    for n in range(1, KNN):
        m = jnp.maximum(m, ws[n])
    es = [jnp.exp(ws[n] - m) for n in range(KNN)]
    ssum = es[0]
    for n in range(1, KNN):
        ssum = ssum + es[n]

    bn = q.shape[0]
    acc = None
    for n in range(KNN):
        wsm = es[n] / ssum
        wtile = jnp.concatenate([wsm] * SHARE, axis=1)
        term = (vg_ref[n] + prs[n]) * wtile
        acc = term if acc is None else acc + term
    out_ref[:] = acc


def _fuse(kg3, vg3, pg3, q, pfat, a1, c1, a2, c2, s1, h1, b1, d1, b2, d2,
          interpret=False):
    nrows = q.shape[0]
    nblk = nrows // _FUSE_BLK
    full = lambda i: (0, 0)
    slab = lambda i: (0, i, 0)
    row = lambda i: (i, 0)
    return pl.pallas_call(
        _fuse_body,
        grid=(nblk,),
        in_specs=[
            pl.BlockSpec((KNN, _FUSE_BLK, MID), slab),
            pl.BlockSpec((KNN, _FUSE_BLK, COUT), slab),
            pl.BlockSpec((KNN, _FUSE_BLK, 16), slab),
            pl.BlockSpec((_FUSE_BLK, MID), row),
            pl.BlockSpec((_FUSE_BLK, 16), row),
            pl.BlockSpec((16, 16), full),
            pl.BlockSpec((1, 16), full),
            pl.BlockSpec((16, COUT), full),
            pl.BlockSpec((1, COUT), full),
            pl.BlockSpec((1, MID), full),
            pl.BlockSpec((1, MID), full),
            pl.BlockSpec((MID, WDIM), full),
            pl.BlockSpec((1, WDIM), full),
            pl.BlockSpec((WDIM, WDIM), full),
            pl.BlockSpec((1, WDIM), full),
        ],
        out_specs=pl.BlockSpec((_FUSE_BLK, COUT), row),
        out_shape=jax.ShapeDtypeStruct((nrows, COUT), jnp.float32),
        interpret=interpret,
    )(kg3, vg3, pg3, q, pfat, a1, c1, a2, c2, s1, h1, b1, d1, b2, d2)


def _prep_params(wq, bq, wk, bk, wv, bv, wp1, bp1, gp, betap, mp, vp,
                 wp2, bp2, g1, beta1, m1, v1, ww1, bw1, g2, beta2, m2, v2,
                 ww2, bw2):
    sp = gp / jnp.sqrt(vp + EPS)
    hp = betap - mp * sp
    s1 = g1 / jnp.sqrt(v1 + EPS)
    h1 = beta1 - m1 * s1
    s2 = g2 / jnp.sqrt(v2 + EPS)
    h2 = beta2 - m2 * s2

    a1 = jnp.zeros((16, 16), jnp.float32).at[:3, :3].set(wp1.T * sp[None, :])
    c1 = jnp.zeros((1, 16), jnp.float32).at[0, :3].set(bp1 * sp + hp)
    a2 = jnp.zeros((16, COUT), jnp.float32).at[:3, :].set(wp2.T)
    c2 = bp2.reshape(1, COUT)

    b1 = ww1.T * s2[None, :]
    d1 = (bw1 * s2 + h2).reshape(1, WDIM)
    b2 = ww2.T
    d2 = bw2.reshape(1, WDIM)
    return (wq.T, bq.reshape(1, MID), wk.T, bk.reshape(1, MID),
            wv.T, bv.reshape(1, COUT),
            a1, c1, a2, c2, s1.reshape(1, MID), h1.reshape(1, MID),
            b1, d1, b2, d2)


def kernel(p, x, o, wq, bq, wk, bk, wv, bv, wp1, bp1, gp, betap, mp, vp,
           wp2, bp2, g1, beta1, m1, v1, ww1, bw1, g2, beta2, m2, v2,
           ww2, bw2):
    (wqT, bq2, wkT, bk2, wvT, bv2, a1, c1, a2, c2, s1, h1,
     b1, d1, b2, d2) = _prep_params(
        wq, bq, wk, bk, wv, bv, wp1, bp1, gp, betap, mp, vp, wp2, bp2,
        g1, beta1, m1, v1, ww1, bw1, g2, beta2, m2, v2, ww2, bw2)

    pT8 = jnp.zeros((8, N), jnp.float32).at[:3, :].set(p.T)
    pfat = jnp.zeros((N, 16), jnp.float32).at[:, :3].set(p)

    H = N // 2
    p8 = pfat[:, :8]
    idx0 = _topk(p8[:H], pT8)
    q, k, v = _qkv(x, wqT, bq2, wkT, bk2, wvT, bv2)
    g0 = _sc_gather(idx0.T.reshape(-1), k, v, pfat)
    idx1 = _topk(p8[H:], pT8)
    g1 = _sc_gather(idx1.T.reshape(-1), k, v, pfat)

    outs = []
    for h, (kg, vg, pg) in enumerate((g0, g1)):
        kg3 = kg.reshape(KNN, H, MID)
        vg3 = vg.reshape(KNN, H, COUT)
        pg3 = pg.reshape(KNN, H, 16)
        sl = slice(h * H, (h + 1) * H)
        outs.append(_fuse(kg3, vg3, pg3, q[sl], pfat[sl], a1, c1, a2, c2,
                          s1, h1, b1, d1, b2, d2))
    return jnp.concatenate(outs, axis=0)


# speedup vs baseline: 1.2735x; 1.2735x over previous
import functools

import jax
import jax.numpy as jnp
from jax import lax
from jax.experimental import pallas as pl
from jax.experimental.pallas import tpu as pltpu
from jax.experimental.pallas import tpu_sc as plsc

N = 8192
KNN = 16
CIN = 256
MID = 256
COUT = 256
SHARE = 8
WDIM = MID // SHARE
EPS = 1e-5

_QKV_BLK = 512
_TOPK_BLK = 256
_FUSE_BLK = 256
_GCHUNK = 128


def _bdot(a, b):
    return jnp.dot(a.astype(jnp.bfloat16), b.astype(jnp.bfloat16),
                   preferred_element_type=jnp.float32)


def _qkv_body(x_ref, wq_ref, bq_ref, wk_ref, bk_ref, wv_ref, bv_ref,
              q_ref, k_ref, v_ref):
    x = x_ref[:].astype(jnp.bfloat16)
    q_ref[:] = jnp.dot(x, wq_ref[:].astype(jnp.bfloat16), preferred_element_type=jnp.float32) + bq_ref[:]
    k_ref[:] = jnp.dot(x, wk_ref[:].astype(jnp.bfloat16), preferred_element_type=jnp.float32) + bk_ref[:]
    v_ref[:] = jnp.dot(x, wv_ref[:].astype(jnp.bfloat16), preferred_element_type=jnp.float32) + bv_ref[:]


def _qkv(x, wqT, bq, wkT, bk, wvT, bv, interpret=False):
    nblk = N // _QKV_BLK
    full = lambda i: (0, 0)
    row = lambda i: (i, 0)
    return pl.pallas_call(
        _qkv_body,
        grid=(nblk,),
        in_specs=[
            pl.BlockSpec((_QKV_BLK, CIN), row),
            pl.BlockSpec((CIN, MID), full),
            pl.BlockSpec((1, MID), full),
            pl.BlockSpec((CIN, MID), full),
            pl.BlockSpec((1, MID), full),
            pl.BlockSpec((CIN, COUT), full),
            pl.BlockSpec((1, COUT), full),
        ],
        out_specs=[
            pl.BlockSpec((_QKV_BLK, MID), row),
            pl.BlockSpec((_QKV_BLK, MID), row),
            pl.BlockSpec((_QKV_BLK, COUT), row),
        ],
        out_shape=[
            jax.ShapeDtypeStruct((N, MID), jnp.float32),
            jax.ShapeDtypeStruct((N, MID), jnp.float32),
            jax.ShapeDtypeStruct((N, COUT), jnp.float32),
        ],
        interpret=interpret,
    )(x, wqT, bq, wkT, bk, wvT, bv)


def _topk_body(p_ref, pT_ref, idx_ref):
    p_blk = p_ref[:]
    px = p_blk[:, 0:1]
    py = p_blk[:, 1:2]
    pz = p_blk[:, 2:3]
    tx = pT_ref[0:1, :]
    ty = pT_ref[1:2, :]
    tz = pT_ref[2:3, :]
    dot = jnp.dot(p_blk.astype(jnp.bfloat16), pT_ref[:].astype(jnp.bfloat16),
                  preferred_element_type=jnp.float32)
    sqb = (px * px + py * py) + pz * pz
    sqa = (tx * tx + ty * ty) + tz * tz
    d2 = (sqb + sqa) - 2.0 * dot

    cols = lax.broadcasted_iota(jnp.int32, d2.shape, 1)
    work = d2
    picks = []
    for n in range(KNN):
        j = jnp.argmin(work, axis=1).reshape(-1, 1).astype(jnp.int32)
        picks.append(j)
        if n < KNN - 1:
            work = jnp.where(cols == j, jnp.float32(jnp.inf), work)
    idx_ref[:] = jnp.concatenate(picks, axis=1)


def _topk(p_rows, pT8, interpret=False):
    nrows = p_rows.shape[0]
    nblk = nrows // _TOPK_BLK
    return pl.pallas_call(
        _topk_body,
        grid=(nblk,),
        in_specs=[
            pl.BlockSpec((_TOPK_BLK, 8), lambda i: (i, 0)),
            pl.BlockSpec((8, N), lambda i: (0, 0)),
        ],
        out_specs=pl.BlockSpec((_TOPK_BLK, KNN), lambda i: (i, 0)),
        out_shape=jax.ShapeDtypeStruct((nrows, KNN), jnp.int32),
        interpret=interpret,
    )(p_rows, pT8)


def _sc_gather(idx_flat, xk, xv, pfat):
    info = plsc.get_sparse_core_info()
    nw = info.num_cores * info.num_subcores
    total = idx_flat.shape[0]
    per_w = total // nw
    nchunk = per_w // _GCHUNK
    mesh = plsc.VectorSubcoreMesh(core_axis_name="c", subcore_axis_name="s")

    @functools.partial(
        pl.kernel,
        mesh=mesh,
        compiler_params=pltpu.CompilerParams(use_tc_tiling_on_sc=False),
        out_type=[
            jax.ShapeDtypeStruct((total, MID), jnp.float32),
            jax.ShapeDtypeStruct((total, COUT), jnp.float32),
            jax.ShapeDtypeStruct((total, 16), jnp.float32),
        ],
        scratch_types=[
            pltpu.VMEM((_GCHUNK,), jnp.int32),
            pltpu.VMEM((_GCHUNK, MID), jnp.float32),
            pltpu.VMEM((_GCHUNK, COUT), jnp.float32),
            pltpu.VMEM((_GCHUNK, 16), jnp.float32),
            pltpu.SemaphoreType.DMA,
            pltpu.SemaphoreType.DMA,
            pltpu.SemaphoreType.DMA,
        ],
    )
    def gather_kernel(idx_hbm, xk_hbm, xv_hbm, pf_hbm,
                      kg_hbm, vg_hbm, pg_hbm,
                      idx_v, kbuf, vbuf, pbuf, sem_k, sem_v, sem_p):
        wid = lax.axis_index("s") * info.num_cores + lax.axis_index("c")
        base = wid * per_w

        def body(c, carry):
            off = base + c * _GCHUNK
            pltpu.sync_copy(idx_hbm.at[pl.ds(off, _GCHUNK)], idx_v)
            ck = pltpu.async_copy(xk_hbm.at[idx_v], kbuf, sem_k)
            cv = pltpu.async_copy(xv_hbm.at[idx_v], vbuf, sem_v)
            cp = pltpu.async_copy(pf_hbm.at[idx_v], pbuf, sem_p)
            ck.wait()
            cv.wait()
            cp.wait()
            pltpu.sync_copy(kbuf, kg_hbm.at[pl.ds(off, _GCHUNK)])
            pltpu.sync_copy(vbuf, vg_hbm.at[pl.ds(off, _GCHUNK)])
            pltpu.sync_copy(pbuf, pg_hbm.at[pl.ds(off, _GCHUNK)])
            return carry

        lax.fori_loop(0, nchunk, body, 0)

    return gather_kernel(idx_flat, xk, xv, pfat)


def _fuse_body(kg_ref, vg_ref, pg_ref, q_ref, pf_ref,
               a1_ref, c1_ref, a2_ref, c2_ref,
               s1_ref, h1_ref, b1_ref, d1_ref, b2_ref, d2_ref,
               out_ref):
    q = q_ref[:]
    pf = pf_ref[:]
    a1 = a1_ref[:]
    c1 = c1_ref[:]
    a2 = a2_ref[:]
    c2 = c2_ref[:]
    s1 = s1_ref[:]
    h1 = h1_ref[:]
    b1 = b1_ref[:]
    d1 = d1_ref[:]
    b2 = b2_ref[:]
    d2 = d2_ref[:]

    prs = []
    ws = []
    for n in range(KNN):
        pr_in = pg_ref[n] - pf
        t = jnp.dot(pr_in, a1, preferred_element_type=jnp.float32) + c1
        t = jnp.maximum(t, 0.0)
        pr_n = jnp.dot(t, a2, preferred_element_type=jnp.float32) + c2
        prs.append(pr_n)
        wn = (kg_ref[n] - q + pr_n) * s1 + h1
        wn = jnp.maximum(wn, 0.0)
        wn = _bdot(wn, b1) + d1
        wn = jnp.maximum(wn, 0.0)
        wn = _bdot(wn, b2) + d2
        ws.append(wn)

    m = ws[0]
    for n in range(1, KNN):
        m = jnp.maximum(m, ws[n])
    es = [jnp.exp(ws[n] - m) for n in range(KNN)]
    ssum = es[0]
    for n in range(1, KNN):
        ssum = ssum + es[n]

    bn = q.shape[0]
    acc = None
    for n in range(KNN):
        wsm = es[n] / ssum
        wtile = jnp.concatenate([wsm] * SHARE, axis=1)
        term = (vg_ref[n] + prs[n]) * wtile
        acc = term if acc is None else acc + term
    out_ref[:] = acc


def _fuse(kg3, vg3, pg3, q, pfat, a1, c1, a2, c2, s1, h1, b1, d1, b2, d2,
          interpret=False):
    nrows = q.shape[0]
    nblk = nrows // _FUSE_BLK
    full = lambda i: (0, 0)
    slab = lambda i: (0, i, 0)
    row = lambda i: (i, 0)
    return pl.pallas_call(
        _fuse_body,
        grid=(nblk,),
        in_specs=[
            pl.BlockSpec((KNN, _FUSE_BLK, MID), slab),
            pl.BlockSpec((KNN, _FUSE_BLK, COUT), slab),
            pl.BlockSpec((KNN, _FUSE_BLK, 16), slab),
            pl.BlockSpec((_FUSE_BLK, MID), row),
            pl.BlockSpec((_FUSE_BLK, 16), row),
            pl.BlockSpec((16, 16), full),
            pl.BlockSpec((1, 16), full),
            pl.BlockSpec((16, COUT), full),
            pl.BlockSpec((1, COUT), full),
            pl.BlockSpec((1, MID), full),
            pl.BlockSpec((1, MID), full),
            pl.BlockSpec((MID, WDIM), full),
            pl.BlockSpec((1, WDIM), full),
            pl.BlockSpec((WDIM, WDIM), full),
            pl.BlockSpec((1, WDIM), full),
        ],
        out_specs=pl.BlockSpec((_FUSE_BLK, COUT), row),
        out_shape=jax.ShapeDtypeStruct((nrows, COUT), jnp.float32),
        interpret=interpret,
    )(kg3, vg3, pg3, q, pfat, a1, c1, a2, c2, s1, h1, b1, d1, b2, d2)


def _prep_params(wq, bq, wk, bk, wv, bv, wp1, bp1, gp, betap, mp, vp,
                 wp2, bp2, g1, beta1, m1, v1, ww1, bw1, g2, beta2, m2, v2,
                 ww2, bw2):
    sp = gp / jnp.sqrt(vp + EPS)
    hp = betap - mp * sp
    s1 = g1 / jnp.sqrt(v1 + EPS)
    h1 = beta1 - m1 * s1
    s2 = g2 / jnp.sqrt(v2 + EPS)
    h2 = beta2 - m2 * s2

    a1 = jnp.zeros((16, 16), jnp.float32).at[:3, :3].set(wp1.T * sp[None, :])
    c1 = jnp.zeros((1, 16), jnp.float32).at[0, :3].set(bp1 * sp + hp)
    a2 = jnp.zeros((16, COUT), jnp.float32).at[:3, :].set(wp2.T)
    c2 = bp2.reshape(1, COUT)

    b1 = ww1.T * s2[None, :]
    d1 = (bw1 * s2 + h2).reshape(1, WDIM)
    b2 = ww2.T
    d2 = bw2.reshape(1, WDIM)
    return (wq.T, bq.reshape(1, MID), wk.T, bk.reshape(1, MID),
            wv.T, bv.reshape(1, COUT),
            a1, c1, a2, c2, s1.reshape(1, MID), h1.reshape(1, MID),
            b1, d1, b2, d2)


def kernel(p, x, o, wq, bq, wk, bk, wv, bv, wp1, bp1, gp, betap, mp, vp,
           wp2, bp2, g1, beta1, m1, v1, ww1, bw1, g2, beta2, m2, v2,
           ww2, bw2):
    (wqT, bq2, wkT, bk2, wvT, bv2, a1, c1, a2, c2, s1, h1,
     b1, d1, b2, d2) = _prep_params(
        wq, bq, wk, bk, wv, bv, wp1, bp1, gp, betap, mp, vp, wp2, bp2,
        g1, beta1, m1, v1, ww1, bw1, g2, beta2, m2, v2, ww2, bw2)

    pT8 = jnp.zeros((8, N), jnp.float32).at[:3, :].set(p.T)
    pfat = jnp.zeros((N, 16), jnp.float32).at[:, :3].set(p)

    H = N // 2
    p8 = pfat[:, :8]
    idx0 = _topk(p8[:H], pT8)
    q, k, v = _qkv(x, wqT, bq2, wkT, bk2, wvT, bv2)
    g0 = _sc_gather(idx0.T.reshape(-1), k, v, pfat)
    idx1 = _topk(p8[H:], pT8)
    g1 = _sc_gather(idx1.T.reshape(-1), k, v, pfat)

    outs = []
    for h, (kg, vg, pg) in enumerate((g0, g1)):
        kg3 = kg.reshape(KNN, H, MID)
        vg3 = vg.reshape(KNN, H, COUT)
        pg3 = pg.reshape(KNN, H, 16)
        sl = slice(h * H, (h + 1) * H)
        outs.append(_fuse(kg3, vg3, pg3, q[sl], pfat[sl], a1, c1, a2, c2,
                          s1, h1, b1, d1, b2, d2))
    return jnp.concatenate(outs, axis=0)
